# baseline (speedup 1.0000x reference)
import math
import functools

import jax
import jax.numpy as jnp
from jax import lax
from jax.experimental import pallas as pl
from jax.experimental.pallas import tpu as pltpu
from jax.experimental.pallas import tpu_sc as plsc

N = 10000
E = 160000
F = 128
L = 20
RC = 5.0
EPS = 1e-5
LOG2 = math.log(2.0)

NBLK = 1000
EBLK = 2000


def _ssp(x):
    return jax.nn.softplus(x) - LOG2


def _mlp1_body(ns_ref, w1_ref, b1_ref, w2_ref, b2_ref, phi_ref):
    h = jnp.dot(ns_ref[...], w1_ref[...], preferred_element_type=jnp.float32)
    h = _ssp(h + b1_ref[...])
    phi_ref[...] = (
        jnp.dot(h, w2_ref[...], preferred_element_type=jnp.float32) + b2_ref[...]
    )


def _mlp1(ns, W1, b1, W2, b2):
    return pl.pallas_call(
        _mlp1_body,
        grid=(N // NBLK,),
        in_specs=[
            pl.BlockSpec((NBLK, F), lambda i: (i, 0)),
            pl.BlockSpec((F, F), lambda i: (0, 0)),
            pl.BlockSpec((1, F), lambda i: (0, 0)),
            pl.BlockSpec((F, 3 * F), lambda i: (0, 0)),
            pl.BlockSpec((1, 3 * F), lambda i: (0, 0)),
        ],
        out_specs=pl.BlockSpec((NBLK, 3 * F), lambda i: (i, 0)),
        out_shape=jax.ShapeDtypeStruct((N, 3 * F), jnp.float32),
    )(ns, W1, b1[None, :], W2, b2[None, :])


def _edge_body(phie_ref, vjc_ref, xs_ref, xd_ref, wmv_ref, bmv_ref,
               dva_ref, ds_ref):
    xs = xs_ref[...]
    xd = xd_ref[...]
    vec = xs - xd
    r2 = jnp.sum(vec * vec, axis=-1, keepdims=True)
    r = jnp.sqrt(r2 + EPS)
    rnorm = jnp.sqrt(r * r + EPS)
    ls = 1.0 + jax.lax.broadcasted_iota(jnp.int32, (1, F), 1).astype(jnp.float32)
    rbf = jnp.sin((math.pi / RC) * (rnorm * ls)) / rnorm
    fc = 0.5 * (jnp.cos(math.pi * (r + EPS) / RC) + 1.0)
    w = fc * (jnp.dot(rbf, wmv_ref[...],
                      preferred_element_type=jnp.float32) + bmv_ref[...])
    msg = phie_ref[...] * w
    v_ = msg[:, 0:F]
    s_ = msg[:, F:2 * F]
    r_ = msg[:, 2 * F:3 * F]
    u = vec / r
    vjc = vjc_ref[...]
    dva_ref[:, 0:F] = vjc[:, 0:F] * v_ + r_ * u[:, 0:1]
    dva_ref[:, F:2 * F] = vjc[:, F:2 * F] * v_ + r_ * u[:, 1:2]
    dva_ref[:, 2 * F:3 * F] = vjc[:, 2 * F:3 * F] * v_ + r_ * u[:, 2:3]
    ds_ref[...] = s_


def _edge_stage(phiE, vjc, xs, xd, WmvP, bmv):
    return pl.pallas_call(
        _edge_body,
        grid=(E // EBLK,),
        in_specs=[
            pl.BlockSpec((EBLK, 3 * F), lambda i: (i, 0)),
            pl.BlockSpec((EBLK, 3 * F), lambda i: (i, 0)),
            pl.BlockSpec((EBLK, 3), lambda i: (i, 0)),
            pl.BlockSpec((EBLK, 3), lambda i: (i, 0)),
            pl.BlockSpec((F, 3 * F), lambda i: (0, 0)),
            pl.BlockSpec((1, 3 * F), lambda i: (0, 0)),
        ],
        out_specs=[
            pl.BlockSpec((EBLK, 3 * F), lambda i: (i, 0)),
            pl.BlockSpec((EBLK, F), lambda i: (i, 0)),
        ],
        out_shape=[
            jax.ShapeDtypeStruct((E, 3 * F), jnp.float32),
            jax.ShapeDtypeStruct((E, F), jnp.float32),
        ],
    )(phiE, vjc, xs, xd, WmvP, bmv[None, :])


def _mlp2_body(vnc_ref, sn_ref, w1_ref, b1_ref, w2_ref, b2_ref, s2_ref):
    vnc = vnc_ref[...]
    nrm = jnp.sqrt(vnc[:, 0:F] ** 2 + vnc[:, F:2 * F] ** 2
                   + vnc[:, 2 * F:3 * F] ** 2 + EPS)
    h = (jnp.dot(nrm, w1_ref[0:F, :], preferred_element_type=jnp.float32)
         + jnp.dot(sn_ref[...], w1_ref[F:2 * F, :],
                   preferred_element_type=jnp.float32))
    h = _ssp(h + b1_ref[...])
    s2_ref[...] = (
        jnp.dot(h, w2_ref[...], preferred_element_type=jnp.float32) + b2_ref[...]
    )


def _mlp2(vnc, s_new, W1, b1, W2, b2):
    return pl.pallas_call(
        _mlp2_body,
        grid=(N // NBLK,),
        in_specs=[
            pl.BlockSpec((NBLK, 3 * F), lambda i: (i, 0)),
            pl.BlockSpec((NBLK, F), lambda i: (i, 0)),
            pl.BlockSpec((2 * F, F), lambda i: (0, 0)),
            pl.BlockSpec((1, F), lambda i: (0, 0)),
            pl.BlockSpec((F, 3 * F), lambda i: (0, 0)),
            pl.BlockSpec((1, 3 * F), lambda i: (0, 0)),
        ],
        out_specs=pl.BlockSpec((NBLK, 3 * F), lambda i: (i, 0)),
        out_shape=jax.ShapeDtypeStruct((N, 3 * F), jnp.float32),
    )(vnc, s_new, W1, b1[None, :], W2, b2[None, :])


def _final_body(vnc_ref, sn_ref, uvs_ref, s2s_ref, deg_ref, vout_ref, sout_ref):
    invd = 1.0 / deg_ref[...]
    uvx = uvs_ref[:, 0:F] * invd
    uvy = uvs_ref[:, F:2 * F] * invd
    uvz = uvs_ref[:, 2 * F:3 * F] * invd
    smean_v = s2s_ref[:, 0:F] * invd
    smean_s = s2s_ref[:, F:2 * F] * invd
    smean_a = s2s_ref[:, 2 * F:3 * F] * invd
    s = uvx * uvx + uvy * uvy + uvz * uvz
    ds2 = s / (s + EPS) * smean_s + smean_a
    vnc = vnc_ref[...]
    vout_ref[:, 0:F] = vnc[:, 0:F] + uvx * smean_v
    vout_ref[:, F:2 * F] = vnc[:, F:2 * F] + uvy * smean_v
    vout_ref[:, 2 * F:3 * F] = vnc[:, 2 * F:3 * F] + uvz * smean_v
    sout_ref[...] = sn_ref[...] + ds2


def _final(vnc, s_new, uv_sum, s2_sum, deg):
    return pl.pallas_call(
        _final_body,
        grid=(N // NBLK,),
        in_specs=[
            pl.BlockSpec((NBLK, 3 * F), lambda i: (i, 0)),
            pl.BlockSpec((NBLK, F), lambda i: (i, 0)),
            pl.BlockSpec((NBLK, 3 * F), lambda i: (i, 0)),
            pl.BlockSpec((NBLK, 3 * F), lambda i: (i, 0)),
            pl.BlockSpec((NBLK, 1), lambda i: (i, 0)),
        ],
        out_specs=[
            pl.BlockSpec((NBLK, 3 * F), lambda i: (i, 0)),
            pl.BlockSpec((NBLK, F), lambda i: (i, 0)),
        ],
        out_shape=[
            jax.ShapeDtypeStruct((N, 3 * F), jnp.float32),
            jax.ShapeDtypeStruct((N, F), jnp.float32),
        ],
    )(vnc, s_new, uv_sum, s2_sum, deg)


NW = 32
NB = 4
EPW = E // NW
PADB = 5120
NQ = N // NB
ACC = 2560
NPAD = NB * ACC
CH = 64

_sc_mesh = plsc.VectorSubcoreMesh(core_axis_name="c", subcore_axis_name="s")
_sc_params = pltpu.CompilerParams(needs_layout_passes=False)


def _partition(src, dst):
    @functools.partial(
        pl.kernel,
        out_type=(
            jax.ShapeDtypeStruct((NB * NW * PADB,), jnp.int32),
            jax.ShapeDtypeStruct((NB * NW * PADB,), jnp.int32),
            jax.ShapeDtypeStruct((NB * NW * PADB,), jnp.int32),
            jax.ShapeDtypeStruct((NB * NW * 16,), jnp.int32),
        ),
        mesh=_sc_mesh,
        compiler_params=_sc_params,
        scratch_types=[
            pltpu.VMEM((EPW,), jnp.int32),
            pltpu.VMEM((EPW,), jnp.int32),
        ] + [pltpu.VMEM((PADB,), jnp.int32)] * (3 * NB) + [
            pltpu.VMEM((16,), jnp.int32),
        ],
    )
    def k(src_hbm, dst_hbm, srcp_hbm, dstl_hbm, eidp_hbm, cnts_hbm,
          src_v, dst_v, *bufs):
        cnt_v = bufs[-1]
        sb_v = bufs[0:NB]
        db_v = bufs[NB:2 * NB]
        eb_v = bufs[2 * NB:3 * NB]
        c = lax.axis_index("c")
        s = lax.axis_index("s")
        wid = c * 16 + s
        base = wid * EPW
        pltpu.sync_copy(src_hbm.at[pl.ds(base, EPW)], src_v)
        pltpu.sync_copy(dst_hbm.at[pl.ds(base, EPW)], dst_v)
        iota = jnp.arange(16, dtype=jnp.int32)
        zeros = jnp.zeros((16,), jnp.int32)
        dump = jnp.full((16,), N, jnp.int32)

        def pre(i, _):
            for b in range(NB):
                sb_v[b][pl.ds(i * 16, 16)] = zeros
                db_v[b][pl.ds(i * 16, 16)] = dump
                eb_v[b][pl.ds(i * 16, 16)] = zeros
            return 0
        lax.fori_loop(0, PADB // 16, pre, 0)

        def body(j, curs):
            off = j * 16
            sv = src_v[pl.ds(off, 16)]
            dv = dst_v[pl.ds(off, 16)]
            valid = (iota + off) < EPW
            eid = base + off + iota
            q = ((dv >= NQ).astype(jnp.int32)
                 + (dv >= 2 * NQ).astype(jnp.int32)
                 + (dv >= 3 * NQ).astype(jnp.int32))
            new = []
            for b in range(NB):
                m = (q == b) & valid
                cs = jnp.cumsum(m.astype(jnp.int32))
                pos = curs[b] + cs - 1
                plsc.store_scatter(sb_v[b], [pos], sv, mask=m)
                plsc.store_scatter(db_v[b], [pos], dv, mask=m)
                plsc.store_scatter(eb_v[b], [pos], eid, mask=m)
                new.append(curs[b] + jnp.max(cs))
            return tuple(new)

        z32 = jnp.int32(0)
        curs = lax.fori_loop(0, (EPW + 15) // 16, body, (z32,) * NB)

        for b in range(NB):
            roff = (b * NW + wid) * PADB
            pltpu.sync_copy(sb_v[b], srcp_hbm.at[pl.ds(roff, PADB)])
            pltpu.sync_copy(db_v[b], dstl_hbm.at[pl.ds(roff, PADB)])
            pltpu.sync_copy(eb_v[b], eidp_hbm.at[pl.ds(roff, PADB)])
            cnt_v[...] = jnp.zeros((16,), jnp.int32) + curs[b]
            pltpu.sync_copy(cnt_v,
                            cnts_hbm.at[pl.ds((b * NW + wid) * 16, 16)])

    return k(src, dst)


def _sc_segsum(table, idxp, dstl, cnts, D):
    @functools.partial(
        pl.kernel,
        out_type=jax.ShapeDtypeStruct((NPAD, D), jnp.float32),
        mesh=_sc_mesh,
        compiler_params=_sc_params,
        scratch_types=[
            pltpu.VMEM((161, D), jnp.float32),
            pltpu.VMEM((PADB,), jnp.int32),
            pltpu.VMEM((PADB,), jnp.int32),
            pltpu.VMEM((PADB,), jnp.int32),
            pltpu.VMEM((PADB,), jnp.int32),
            pltpu.VMEM((CH, D), jnp.float32),
            pltpu.VMEM((16,), jnp.int32),
            pltpu.SemaphoreType.DMA,
        ],
    )
    def k(table_hbm, idxp_hbm, dstl_hbm, cnts_hbm, out_hbm,
          acc, idx_v, dst_v, pend_i, pend_r, rows_v, cntv, sem):
        c = lax.axis_index("c")
        s = lax.axis_index("s")
        iota = jnp.arange(16, dtype=jnp.int32)
        zf = jnp.zeros((16,), jnp.float32)

        def pf(i, _):
            pend_i[pl.ds(i * 16, 16)] = jnp.zeros((16,), jnp.int32)
            pend_r[pl.ds(i * 16, 16)] = jnp.full((16,), 160, jnp.int32)
            return 0
        lax.fori_loop(0, PADB // 16, pf, 0)

        for half in range(2):
            b = c * 2 + half
            lo = s * 160

            def za(i, _):
                for jj in range(D // 16):
                    acc[i, pl.ds(jj * 16, 16)] = zf
                return 0
            lax.fori_loop(0, 161, za, 0)

            def region(r, _):
                roff = (b * NW + r) * PADB
                pltpu.sync_copy(
                    cnts_hbm.at[pl.ds((b * NW + r) * 16, 16)], cntv)
                cnt = jnp.max(cntv[...])

                npc = lax.shift_right_logical(cnt + 1023, 10)

                def cp(kk, _):
                    pltpu.sync_copy(
                        idxp_hbm.at[pl.ds(roff + kk * 1024, 1024)],
                        idx_v.at[pl.ds(kk * 1024, 1024)])
                    pltpu.sync_copy(
                        dstl_hbm.at[pl.ds(roff + kk * 1024, 1024)],
                        dst_v.at[pl.ds(kk * 1024, 1024)])
                    return 0
                lax.fori_loop(0, npc, cp, 0)

                nch = lax.shift_right_logical(cnt + 15, 4)

                def scan(j, pcur):
                    dv = dst_v[pl.ds(j * 16, 16)]
                    iv = idx_v[pl.ds(j * 16, 16)]
                    local = dv - b * NQ
                    m = (local >= lo) & (local < lo + 160)
                    cs = jnp.cumsum(m.astype(jnp.int32))
                    pos = pcur + cs - 1
                    plsc.store_scatter(pend_i, [pos], iv, mask=m)
                    plsc.store_scatter(pend_r, [pos], local - lo, mask=m)
                    return pcur + jnp.max(cs)
                pcur = lax.fori_loop(0, nch, scan, jnp.int32(0))

                nfl = lax.shift_right_logical(pcur + CH - 1, 6)

                def flush(f, _):
                    pltpu.async_copy(
                        table_hbm.at[pend_i.at[pl.ds(f * CH, CH)]],
                        rows_v, sem).wait()
                    nacc = jnp.minimum(pcur - f * CH, CH)

                    def accb(i, _):
                        p = f * CH + i
                        grp = lax.shift_right_logical(p, 4) * 16
                        rv = pend_r[pl.ds(grp, 16)]
                        lane = p & 15
                        row = jnp.max(jnp.where(iota == lane, rv, 0))
                        for jj in range(D // 16):
                            plsc.addupdate(
                                acc.at[row, pl.ds(jj * 16, 16)],
                                rows_v[i, pl.ds(jj * 16, 16)])
                        return 0
                    lax.fori_loop(0, nacc, accb, 0)

                    def rd(i, _):
                        pend_i[pl.ds(f * CH + i * 16, 16)] = (
                            jnp.zeros((16,), jnp.int32))
                        pend_r[pl.ds(f * CH + i * 16, 16)] = (
                            jnp.full((16,), 160, jnp.int32))
                        return 0
                    lax.fori_loop(0, CH // 16, rd, 0)
                    return 0
                lax.fori_loop(0, nfl, flush, 0)
                return 0
            lax.fori_loop(0, NW, region, 0)

            pltpu.sync_copy(acc.at[pl.ds(0, 160)],
                            out_hbm.at[pl.ds(b * ACC + lo, 160)])

    return k(table, idxp, dstl, cnts)


def _unpad_nodes(arr):
    return jnp.concatenate(
        [arr[b * ACC:b * ACC + NQ] for b in range(NB)], axis=0)


def kernel(nv, ns, x, edge_index,
           Wms1, bms1, Wms2, bms2, Wmv, bmv, Wus1, bus1, Wus2, bus2):
    src = edge_index[0]
    dst = edge_index[1]

    phi = _mlp1(ns, Wms1, bms1, Wms2, bms2)

    WmvP = jnp.zeros((F, 3 * F), jnp.float32).at[0:L, :].set(Wmv)

    vnc0 = jnp.transpose(nv, (0, 2, 1)).reshape(N, 3 * F)

    phiE = jnp.take(phi, src, axis=0)
    vjc = jnp.take(vnc0, src, axis=0)
    xs = jnp.take(x, src, axis=0)
    xd = jnp.take(x, dst, axis=0)

    dva, ds_e = _edge_stage(phiE, vjc, xs, xd, WmvP, bmv)

    dv = jax.ops.segment_sum(dva, dst, num_segments=N)
    ds = jax.ops.segment_sum(ds_e, dst, num_segments=N)
    vnc = vnc0 + dv
    s_new = ns + ds

    s2 = _mlp2(vnc, s_new, Wus1, bus1, Wus2, bus2)

    ones = jnp.ones((E,), jnp.float32)
    deg = jnp.maximum(jax.ops.segment_sum(ones, dst, num_segments=N), 1.0)
    srcp, dstl, eidp, cnts = _partition(src, dst)
    del eidp
    uv_sum = _unpad_nodes(_sc_segsum(vnc, srcp, dstl, cnts, 3 * F))
    s2_sum = _unpad_nodes(_sc_segsum(s2, srcp, dstl, cnts, 3 * F))

    vout_c, sout = _final(vnc, s_new, uv_sum, s2_sum, deg[:, None])
    vout = jnp.transpose(vout_c.reshape(N, 3, F), (0, 2, 1))
    return (vout, sout)


# speedup vs baseline: 1.7837x; 1.7837x over previous
import math
import functools

import jax
import jax.numpy as jnp
from jax import lax
from jax.experimental import pallas as pl
from jax.experimental.pallas import tpu as pltpu
from jax.experimental.pallas import tpu_sc as plsc

N = 10000
E = 160000
F = 128
L = 20
RC = 5.0
EPS = 1e-5
LOG2 = math.log(2.0)

NBLK = 1000
EBLK = 2000


def _ssp(x):
    return jax.nn.softplus(x) - LOG2


def _mlp1_body(ns_ref, w1_ref, b1_ref, w2_ref, b2_ref, phi_ref):
    h = jnp.dot(ns_ref[...], w1_ref[...], preferred_element_type=jnp.float32)
    h = _ssp(h + b1_ref[...])
    phi_ref[...] = (
        jnp.dot(h, w2_ref[...], preferred_element_type=jnp.float32) + b2_ref[...]
    )


def _mlp1(ns, W1, b1, W2, b2):
    return pl.pallas_call(
        _mlp1_body,
        grid=(N // NBLK,),
        in_specs=[
            pl.BlockSpec((NBLK, F), lambda i: (i, 0)),
            pl.BlockSpec((F, F), lambda i: (0, 0)),
            pl.BlockSpec((1, F), lambda i: (0, 0)),
            pl.BlockSpec((F, 3 * F), lambda i: (0, 0)),
            pl.BlockSpec((1, 3 * F), lambda i: (0, 0)),
        ],
        out_specs=pl.BlockSpec((NBLK, 3 * F), lambda i: (i, 0)),
        out_shape=jax.ShapeDtypeStruct((N, 3 * F), jnp.float32),
    )(ns, W1, b1[None, :], W2, b2[None, :])


def _edge_body(phie_ref, vjc_ref, xs_ref, xd_ref, wmv_ref, bmv_ref,
               dva_ref, ds_ref):
    xs = xs_ref[...]
    xd = xd_ref[...]
    vec = xs - xd
    r2 = jnp.sum(vec * vec, axis=-1, keepdims=True)
    r = jnp.sqrt(r2 + EPS)
    rnorm = jnp.sqrt(r * r + EPS)
    ls = 1.0 + jax.lax.broadcasted_iota(jnp.int32, (1, F), 1).astype(jnp.float32)
    rbf = jnp.sin((math.pi / RC) * (rnorm * ls)) / rnorm
    fc = 0.5 * (jnp.cos(math.pi * (r + EPS) / RC) + 1.0)
    w = fc * (jnp.dot(rbf, wmv_ref[...],
                      preferred_element_type=jnp.float32) + bmv_ref[...])
    msg = phie_ref[...] * w
    v_ = msg[:, 0:F]
    s_ = msg[:, F:2 * F]
    r_ = msg[:, 2 * F:3 * F]
    u = vec / r
    vjc = vjc_ref[...]
    dva_ref[:, 0:F] = vjc[:, 0:F] * v_ + r_ * u[:, 0:1]
    dva_ref[:, F:2 * F] = vjc[:, F:2 * F] * v_ + r_ * u[:, 1:2]
    dva_ref[:, 2 * F:3 * F] = vjc[:, 2 * F:3 * F] * v_ + r_ * u[:, 2:3]
    ds_ref[...] = s_


def _edge_stage(phiE, vjc, xs, xd, WmvP, bmv):
    return pl.pallas_call(
        _edge_body,
        grid=(E // EBLK,),
        in_specs=[
            pl.BlockSpec((EBLK, 3 * F), lambda i: (i, 0)),
            pl.BlockSpec((EBLK, 3 * F), lambda i: (i, 0)),
            pl.BlockSpec((EBLK, 3), lambda i: (i, 0)),
            pl.BlockSpec((EBLK, 3), lambda i: (i, 0)),
            pl.BlockSpec((F, 3 * F), lambda i: (0, 0)),
            pl.BlockSpec((1, 3 * F), lambda i: (0, 0)),
        ],
        out_specs=[
            pl.BlockSpec((EBLK, 3 * F), lambda i: (i, 0)),
            pl.BlockSpec((EBLK, F), lambda i: (i, 0)),
        ],
        out_shape=[
            jax.ShapeDtypeStruct((E, 3 * F), jnp.float32),
            jax.ShapeDtypeStruct((E, F), jnp.float32),
        ],
    )(phiE, vjc, xs, xd, WmvP, bmv[None, :])


def _mlp2_body(vnc_ref, sn_ref, w1_ref, b1_ref, w2_ref, b2_ref, s2_ref):
    vnc = vnc_ref[...]
    nrm = jnp.sqrt(vnc[:, 0:F] ** 2 + vnc[:, F:2 * F] ** 2
                   + vnc[:, 2 * F:3 * F] ** 2 + EPS)
    h = (jnp.dot(nrm, w1_ref[0:F, :], preferred_element_type=jnp.float32)
         + jnp.dot(sn_ref[...], w1_ref[F:2 * F, :],
                   preferred_element_type=jnp.float32))
    h = _ssp(h + b1_ref[...])
    s2_ref[...] = (
        jnp.dot(h, w2_ref[...], preferred_element_type=jnp.float32) + b2_ref[...]
    )


def _mlp2(vnc, s_new, W1, b1, W2, b2):
    return pl.pallas_call(
        _mlp2_body,
        grid=(N // NBLK,),
        in_specs=[
            pl.BlockSpec((NBLK, 3 * F), lambda i: (i, 0)),
            pl.BlockSpec((NBLK, F), lambda i: (i, 0)),
            pl.BlockSpec((2 * F, F), lambda i: (0, 0)),
            pl.BlockSpec((1, F), lambda i: (0, 0)),
            pl.BlockSpec((F, 3 * F), lambda i: (0, 0)),
            pl.BlockSpec((1, 3 * F), lambda i: (0, 0)),
        ],
        out_specs=pl.BlockSpec((NBLK, 3 * F), lambda i: (i, 0)),
        out_shape=jax.ShapeDtypeStruct((N, 3 * F), jnp.float32),
    )(vnc, s_new, W1, b1[None, :], W2, b2[None, :])


def _final_body(vnc_ref, sn_ref, uvs_ref, s2s_ref, deg_ref, vout_ref, sout_ref):
    invd = 1.0 / deg_ref[...]
    uvx = uvs_ref[:, 0:F] * invd
    uvy = uvs_ref[:, F:2 * F] * invd
    uvz = uvs_ref[:, 2 * F:3 * F] * invd
    smean_v = s2s_ref[:, 0:F] * invd
    smean_s = s2s_ref[:, F:2 * F] * invd
    smean_a = s2s_ref[:, 2 * F:3 * F] * invd
    s = uvx * uvx + uvy * uvy + uvz * uvz
    ds2 = s / (s + EPS) * smean_s + smean_a
    vnc = vnc_ref[...]
    vout_ref[:, 0:F] = vnc[:, 0:F] + uvx * smean_v
    vout_ref[:, F:2 * F] = vnc[:, F:2 * F] + uvy * smean_v
    vout_ref[:, 2 * F:3 * F] = vnc[:, 2 * F:3 * F] + uvz * smean_v
    sout_ref[...] = sn_ref[...] + ds2


def _final(vnc, s_new, uv_sum, s2_sum, deg):
    return pl.pallas_call(
        _final_body,
        grid=(N // NBLK,),
        in_specs=[
            pl.BlockSpec((NBLK, 3 * F), lambda i: (i, 0)),
            pl.BlockSpec((NBLK, F), lambda i: (i, 0)),
            pl.BlockSpec((NBLK, 3 * F), lambda i: (i, 0)),
            pl.BlockSpec((NBLK, 3 * F), lambda i: (i, 0)),
            pl.BlockSpec((NBLK, 1), lambda i: (i, 0)),
        ],
        out_specs=[
            pl.BlockSpec((NBLK, 3 * F), lambda i: (i, 0)),
            pl.BlockSpec((NBLK, F), lambda i: (i, 0)),
        ],
        out_shape=[
            jax.ShapeDtypeStruct((N, 3 * F), jnp.float32),
            jax.ShapeDtypeStruct((N, F), jnp.float32),
        ],
    )(vnc, s_new, uv_sum, s2_sum, deg)


NW = 32
NB = 4
EPW = E // NW
PADB = 5120
NQ = N // NB
ACC = 2560
NPAD = NB * ACC
CH = 32

_sc_mesh = plsc.VectorSubcoreMesh(core_axis_name="c", subcore_axis_name="s")
_sc_params = pltpu.CompilerParams(needs_layout_passes=False)


def _partition(src, dst):
    @functools.partial(
        pl.kernel,
        out_type=(
            jax.ShapeDtypeStruct((NB * NW * PADB,), jnp.int32),
            jax.ShapeDtypeStruct((NB * NW * PADB,), jnp.int32),
            jax.ShapeDtypeStruct((NB * NW * PADB,), jnp.int32),
            jax.ShapeDtypeStruct((NB * NW * 16,), jnp.int32),
        ),
        mesh=_sc_mesh,
        compiler_params=_sc_params,
        scratch_types=[
            pltpu.VMEM((EPW,), jnp.int32),
            pltpu.VMEM((EPW,), jnp.int32),
        ] + [pltpu.VMEM((PADB,), jnp.int32)] * (3 * NB) + [
            pltpu.VMEM((16,), jnp.int32),
        ],
    )
    def k(src_hbm, dst_hbm, srcp_hbm, dstl_hbm, eidp_hbm, cnts_hbm,
          src_v, dst_v, *bufs):
        cnt_v = bufs[-1]
        sb_v = bufs[0:NB]
        db_v = bufs[NB:2 * NB]
        eb_v = bufs[2 * NB:3 * NB]
        c = lax.axis_index("c")
        s = lax.axis_index("s")
        wid = c * 16 + s
        base = wid * EPW
        pltpu.sync_copy(src_hbm.at[pl.ds(base, EPW)], src_v)
        pltpu.sync_copy(dst_hbm.at[pl.ds(base, EPW)], dst_v)
        iota = jnp.arange(16, dtype=jnp.int32)
        zeros = jnp.zeros((16,), jnp.int32)
        dump = jnp.full((16,), N, jnp.int32)

        def pre(i, _):
            for b in range(NB):
                sb_v[b][pl.ds(i * 16, 16)] = zeros
                db_v[b][pl.ds(i * 16, 16)] = dump
                eb_v[b][pl.ds(i * 16, 16)] = zeros
            return 0
        lax.fori_loop(0, PADB // 16, pre, 0)

        def body(j, curs):
            off = j * 16
            sv = src_v[pl.ds(off, 16)]
            dv = dst_v[pl.ds(off, 16)]
            valid = (iota + off) < EPW
            eid = base + off + iota
            q = ((dv >= NQ).astype(jnp.int32)
                 + (dv >= 2 * NQ).astype(jnp.int32)
                 + (dv >= 3 * NQ).astype(jnp.int32))
            new = []
            for b in range(NB):
                m = (q == b) & valid
                cs = jnp.cumsum(m.astype(jnp.int32))
                pos = curs[b] + cs - 1
                plsc.store_scatter(sb_v[b], [pos], sv, mask=m)
                plsc.store_scatter(db_v[b], [pos], dv, mask=m)
                plsc.store_scatter(eb_v[b], [pos], eid, mask=m)
                new.append(curs[b] + jnp.max(cs))
            return tuple(new)

        z32 = jnp.int32(0)
        curs = lax.fori_loop(0, (EPW + 15) // 16, body, (z32,) * NB)

        for b in range(NB):
            roff = (b * NW + wid) * PADB
            pltpu.sync_copy(sb_v[b], srcp_hbm.at[pl.ds(roff, PADB)])
            pltpu.sync_copy(db_v[b], dstl_hbm.at[pl.ds(roff, PADB)])
            pltpu.sync_copy(eb_v[b], eidp_hbm.at[pl.ds(roff, PADB)])
            cnt_v[...] = jnp.zeros((16,), jnp.int32) + curs[b]
            pltpu.sync_copy(cnt_v,
                            cnts_hbm.at[pl.ds((b * NW + wid) * 16, 16)])

    return k(src, dst)


def _sc_segsum(table, idxp, dstl, cnts, D):
    @functools.partial(
        pl.kernel,
        out_type=jax.ShapeDtypeStruct((NPAD, D), jnp.float32),
        mesh=_sc_mesh,
        compiler_params=_sc_params,
        scratch_types=[
            pltpu.VMEM((161, D), jnp.float32),
            pltpu.VMEM((PADB,), jnp.int32),
            pltpu.VMEM((PADB,), jnp.int32),
            pltpu.VMEM((PADB,), jnp.int32),
            pltpu.VMEM((PADB,), jnp.int32),
            pltpu.VMEM((CH, D), jnp.float32),
            pltpu.VMEM((CH, D), jnp.float32),
            pltpu.VMEM((16,), jnp.int32),
            pltpu.SemaphoreType.DMA,
            pltpu.SemaphoreType.DMA,
        ],
    )
    def k(table_hbm, idxp_hbm, dstl_hbm, cnts_hbm, out_hbm,
          acc, idx_v, dst_v, pend_i, pend_r, rows_a, rows_b, cntv,
          sema, semb):
        c = lax.axis_index("c")
        s = lax.axis_index("s")
        iota = jnp.arange(16, dtype=jnp.int32)
        zf = jnp.zeros((16,), jnp.float32)

        def pf(i, _):
            pend_i[pl.ds(i * 16, 16)] = jnp.zeros((16,), jnp.int32)
            pend_r[pl.ds(i * 16, 16)] = jnp.full((16,), 160, jnp.int32)
            return 0
        lax.fori_loop(0, PADB // 16, pf, 0)

        for half in range(2):
            b = c * 2 + half
            lo = s * 160

            def za(i, _):
                for jj in range(D // 16):
                    acc[i, pl.ds(jj * 16, 16)] = zf
                return 0
            lax.fori_loop(0, 161, za, 0)

            def region(r, _):
                roff = (b * NW + r) * PADB
                pltpu.sync_copy(
                    cnts_hbm.at[pl.ds((b * NW + r) * 16, 16)], cntv)
                cnt = jnp.max(cntv[...])

                npc = lax.shift_right_logical(cnt + 1023, 10)

                def cp(kk, _):
                    pltpu.sync_copy(
                        idxp_hbm.at[pl.ds(roff + kk * 1024, 1024)],
                        idx_v.at[pl.ds(kk * 1024, 1024)])
                    pltpu.sync_copy(
                        dstl_hbm.at[pl.ds(roff + kk * 1024, 1024)],
                        dst_v.at[pl.ds(kk * 1024, 1024)])
                    return 0
                lax.fori_loop(0, npc, cp, 0)

                nch = lax.shift_right_logical(cnt + 15, 4)

                def scan(j, pcur):
                    dv = dst_v[pl.ds(j * 16, 16)]
                    iv = idx_v[pl.ds(j * 16, 16)]
                    local = dv - b * NQ
                    m = (local >= lo) & (local < lo + 160)
                    cs = jnp.cumsum(m.astype(jnp.int32))
                    pos = pcur + cs - 1
                    plsc.store_scatter(pend_i, [pos], iv, mask=m)
                    plsc.store_scatter(pend_r, [pos], local - lo, mask=m)
                    return pcur + jnp.max(cs)
                pcur = lax.fori_loop(0, nch, scan, jnp.int32(0))

                nfl = jnp.maximum(
                    lax.shift_right_logical(pcur + CH - 1,
                                            CH.bit_length() - 1), 1)
                npr = lax.shift_right_logical(nfl + 1, 1)

                def accum(f, rows_v):
                    nacc = jnp.minimum(pcur - f * CH, CH)

                    def accb(i, _):
                        p = f * CH + i
                        grp = lax.shift_right_logical(p, 4) * 16
                        rv = pend_r[pl.ds(grp, 16)]
                        lane = p & 15
                        row = jnp.max(jnp.where(iota == lane, rv, 0))
                        for jj in range(D // 16):
                            plsc.addupdate(
                                acc.at[row, pl.ds(jj * 16, 16)],
                                rows_v[i, pl.ds(jj * 16, 16)])
                        return 0
                    lax.fori_loop(0, nacc, accb, 0)

                def pair(g, _):
                    f0 = 2 * g
                    f1 = f0 + 1
                    f1c = jnp.minimum(f1, nfl - 1)
                    cpa = pltpu.async_copy(
                        table_hbm.at[pend_i.at[pl.ds(f0 * CH, CH)]],
                        rows_a, sema)
                    cpb = pltpu.async_copy(
                        table_hbm.at[pend_i.at[pl.ds(f1c * CH, CH)]],
                        rows_b, semb)
                    cpa.wait()
                    accum(f0, rows_a)
                    cpb.wait()
                    accum(f1, rows_b)
                    return 0
                lax.fori_loop(0, npr, pair, 0)
                return 0
            lax.fori_loop(0, NW, region, 0)

            pltpu.sync_copy(acc.at[pl.ds(0, 160)],
                            out_hbm.at[pl.ds(b * ACC + lo, 160)])

    return k(table, idxp, dstl, cnts)


def _unpad_nodes(arr):
    return jnp.concatenate(
        [arr[b * ACC:b * ACC + NQ] for b in range(NB)], axis=0)


def kernel(nv, ns, x, edge_index,
           Wms1, bms1, Wms2, bms2, Wmv, bmv, Wus1, bus1, Wus2, bus2):
    src = edge_index[0]
    dst = edge_index[1]

    phi = _mlp1(ns, Wms1, bms1, Wms2, bms2)

    WmvP = jnp.zeros((F, 3 * F), jnp.float32).at[0:L, :].set(Wmv)

    vnc0 = jnp.transpose(nv, (0, 2, 1)).reshape(N, 3 * F)

    phiE = jnp.take(phi, src, axis=0)
    vjc = jnp.take(vnc0, src, axis=0)
    xs = jnp.take(x, src, axis=0)
    xd = jnp.take(x, dst, axis=0)

    dva, ds_e = _edge_stage(phiE, vjc, xs, xd, WmvP, bmv)

    dv = jax.ops.segment_sum(dva, dst, num_segments=N)
    ds = jax.ops.segment_sum(ds_e, dst, num_segments=N)
    vnc = vnc0 + dv
    s_new = ns + ds

    s2 = _mlp2(vnc, s_new, Wus1, bus1, Wus2, bus2)

    ones = jnp.ones((E,), jnp.float32)
    deg = jnp.maximum(jax.ops.segment_sum(ones, dst, num_segments=N), 1.0)
    srcp, dstl, eidp, cnts = _partition(src, dst)
    del eidp
    uv_sum = _unpad_nodes(_sc_segsum(vnc, srcp, dstl, cnts, 3 * F))
    s2_sum = _unpad_nodes(_sc_segsum(s2, srcp, dstl, cnts, 3 * F))

    vout_c, sout = _final(vnc, s_new, uv_sum, s2_sum, deg[:, None])
    vout = jnp.transpose(vout_c.reshape(N, 3, F), (0, 2, 1))
    return (vout, sout)


# speedup vs baseline: 2.1160x; 1.1863x over previous
import math
import functools

import jax
import jax.numpy as jnp
from jax import lax
from jax.experimental import pallas as pl
from jax.experimental.pallas import tpu as pltpu
from jax.experimental.pallas import tpu_sc as plsc

N = 10000
E = 160000
F = 128
L = 20
RC = 5.0
EPS = 1e-5
LOG2 = math.log(2.0)

NBLK = 1000
EBLK = 2000


def _ssp(x):
    return jax.nn.softplus(x) - LOG2


def _mlp1_body(ns_ref, w1_ref, b1_ref, w2_ref, b2_ref, phi_ref):
    h = jnp.dot(ns_ref[...], w1_ref[...], preferred_element_type=jnp.float32)
    h = _ssp(h + b1_ref[...])
    phi_ref[...] = (
        jnp.dot(h, w2_ref[...], preferred_element_type=jnp.float32) + b2_ref[...]
    )


def _mlp1(ns, W1, b1, W2, b2):
    return pl.pallas_call(
        _mlp1_body,
        grid=(N // NBLK,),
        in_specs=[
            pl.BlockSpec((NBLK, F), lambda i: (i, 0)),
            pl.BlockSpec((F, F), lambda i: (0, 0)),
            pl.BlockSpec((1, F), lambda i: (0, 0)),
            pl.BlockSpec((F, 3 * F), lambda i: (0, 0)),
            pl.BlockSpec((1, 3 * F), lambda i: (0, 0)),
        ],
        out_specs=pl.BlockSpec((NBLK, 3 * F), lambda i: (i, 0)),
        out_shape=jax.ShapeDtypeStruct((N, 3 * F), jnp.float32),
    )(ns, W1, b1[None, :], W2, b2[None, :])


def _edge_body(phivj_ref, xs_ref, xd_ref, wmv_ref, bmv_ref,
               dva_ref, ds_ref):
    xs = xs_ref[...]
    xd = xd_ref[...]
    vec = xs - xd
    r2 = jnp.sum(vec * vec, axis=-1, keepdims=True)
    r = jnp.sqrt(r2 + EPS)
    rnorm = jnp.sqrt(r * r + EPS)
    ls = 1.0 + jax.lax.broadcasted_iota(jnp.int32, (1, F), 1).astype(jnp.float32)
    rbf = jnp.sin((math.pi / RC) * (rnorm * ls)) / rnorm
    fc = 0.5 * (jnp.cos(math.pi * (r + EPS) / RC) + 1.0)
    w = fc * (jnp.dot(rbf, wmv_ref[...],
                      preferred_element_type=jnp.float32) + bmv_ref[...])
    msg = phivj_ref[:, 0:3 * F] * w
    v_ = msg[:, 0:F]
    s_ = msg[:, F:2 * F]
    r_ = msg[:, 2 * F:3 * F]
    u = vec / r
    vjc = phivj_ref[:, 3 * F:6 * F]
    dva_ref[:, 0:F] = vjc[:, 0:F] * v_ + r_ * u[:, 0:1]
    dva_ref[:, F:2 * F] = vjc[:, F:2 * F] * v_ + r_ * u[:, 1:2]
    dva_ref[:, 2 * F:3 * F] = vjc[:, 2 * F:3 * F] * v_ + r_ * u[:, 2:3]
    ds_ref[...] = s_


def _edge_stage(phivjE, xs, xd, WmvP, bmv):
    return pl.pallas_call(
        _edge_body,
        grid=(E // EBLK,),
        in_specs=[
            pl.BlockSpec((EBLK, 6 * F), lambda i: (i, 0)),
            pl.BlockSpec((EBLK, 3), lambda i: (i, 0)),
            pl.BlockSpec((EBLK, 3), lambda i: (i, 0)),
            pl.BlockSpec((F, 3 * F), lambda i: (0, 0)),
            pl.BlockSpec((1, 3 * F), lambda i: (0, 0)),
        ],
        out_specs=[
            pl.BlockSpec((EBLK, 3 * F), lambda i: (i, 0)),
            pl.BlockSpec((EBLK, F), lambda i: (i, 0)),
        ],
        out_shape=[
            jax.ShapeDtypeStruct((E, 3 * F), jnp.float32),
            jax.ShapeDtypeStruct((E, F), jnp.float32),
        ],
    )(phivjE, xs, xd, WmvP, bmv[None, :])


def _mlp2_body(vnc_ref, sn_ref, w1_ref, b1_ref, w2_ref, b2_ref, s2_ref):
    vnc = vnc_ref[...]
    nrm = jnp.sqrt(vnc[:, 0:F] ** 2 + vnc[:, F:2 * F] ** 2
                   + vnc[:, 2 * F:3 * F] ** 2 + EPS)
    h = (jnp.dot(nrm, w1_ref[0:F, :], preferred_element_type=jnp.float32)
         + jnp.dot(sn_ref[...], w1_ref[F:2 * F, :],
                   preferred_element_type=jnp.float32))
    h = _ssp(h + b1_ref[...])
    s2_ref[...] = (
        jnp.dot(h, w2_ref[...], preferred_element_type=jnp.float32) + b2_ref[...]
    )


def _mlp2(vnc, s_new, W1, b1, W2, b2):
    return pl.pallas_call(
        _mlp2_body,
        grid=(N // NBLK,),
        in_specs=[
            pl.BlockSpec((NBLK, 3 * F), lambda i: (i, 0)),
            pl.BlockSpec((NBLK, F), lambda i: (i, 0)),
            pl.BlockSpec((2 * F, F), lambda i: (0, 0)),
            pl.BlockSpec((1, F), lambda i: (0, 0)),
            pl.BlockSpec((F, 3 * F), lambda i: (0, 0)),
            pl.BlockSpec((1, 3 * F), lambda i: (0, 0)),
        ],
        out_specs=pl.BlockSpec((NBLK, 3 * F), lambda i: (i, 0)),
        out_shape=jax.ShapeDtypeStruct((N, 3 * F), jnp.float32),
    )(vnc, s_new, W1, b1[None, :], W2, b2[None, :])


def _final_body(vnc_ref, sn_ref, uvs_ref, s2s_ref, deg_ref, vout_ref, sout_ref):
    invd = 1.0 / deg_ref[...]
    uvx = uvs_ref[:, 0:F] * invd
    uvy = uvs_ref[:, F:2 * F] * invd
    uvz = uvs_ref[:, 2 * F:3 * F] * invd
    smean_v = s2s_ref[:, 0:F] * invd
    smean_s = s2s_ref[:, F:2 * F] * invd
    smean_a = s2s_ref[:, 2 * F:3 * F] * invd
    s = uvx * uvx + uvy * uvy + uvz * uvz
    ds2 = s / (s + EPS) * smean_s + smean_a
    vnc = vnc_ref[...]
    vout_ref[:, 0:F] = vnc[:, 0:F] + uvx * smean_v
    vout_ref[:, F:2 * F] = vnc[:, F:2 * F] + uvy * smean_v
    vout_ref[:, 2 * F:3 * F] = vnc[:, 2 * F:3 * F] + uvz * smean_v
    sout_ref[...] = sn_ref[...] + ds2


def _final(vnc, s_new, uv_sum, s2_sum, deg):
    return pl.pallas_call(
        _final_body,
        grid=(N // NBLK,),
        in_specs=[
            pl.BlockSpec((NBLK, 3 * F), lambda i: (i, 0)),
            pl.BlockSpec((NBLK, F), lambda i: (i, 0)),
            pl.BlockSpec((NBLK, 3 * F), lambda i: (i, 0)),
            pl.BlockSpec((NBLK, 3 * F), lambda i: (i, 0)),
            pl.BlockSpec((NBLK, 1), lambda i: (i, 0)),
        ],
        out_specs=[
            pl.BlockSpec((NBLK, 3 * F), lambda i: (i, 0)),
            pl.BlockSpec((NBLK, F), lambda i: (i, 0)),
        ],
        out_shape=[
            jax.ShapeDtypeStruct((N, 3 * F), jnp.float32),
            jax.ShapeDtypeStruct((N, F), jnp.float32),
        ],
    )(vnc, s_new, uv_sum, s2_sum, deg)


NW = 32
NB = 4
EPW = E // NW
PADB = 5120
NQ = N // NB
ACC = 2560
NPAD = NB * ACC
CH = 32

_sc_mesh = plsc.VectorSubcoreMesh(core_axis_name="c", subcore_axis_name="s")
_sc_params = pltpu.CompilerParams(needs_layout_passes=False)


def _partition(src, dst):
    @functools.partial(
        pl.kernel,
        out_type=(
            jax.ShapeDtypeStruct((NB * NW * PADB,), jnp.int32),
            jax.ShapeDtypeStruct((NB * NW * PADB,), jnp.int32),
            jax.ShapeDtypeStruct((NB * NW * PADB,), jnp.int32),
            jax.ShapeDtypeStruct((NB * NW * 16,), jnp.int32),
        ),
        mesh=_sc_mesh,
        compiler_params=_sc_params,
        scratch_types=[
            pltpu.VMEM((EPW,), jnp.int32),
            pltpu.VMEM((EPW,), jnp.int32),
        ] + [pltpu.VMEM((PADB,), jnp.int32)] * (3 * NB) + [
            pltpu.VMEM((16,), jnp.int32),
        ],
    )
    def k(src_hbm, dst_hbm, srcp_hbm, dstl_hbm, eidp_hbm, cnts_hbm,
          src_v, dst_v, *bufs):
        cnt_v = bufs[-1]
        sb_v = bufs[0:NB]
        db_v = bufs[NB:2 * NB]
        eb_v = bufs[2 * NB:3 * NB]
        c = lax.axis_index("c")
        s = lax.axis_index("s")
        wid = c * 16 + s
        base = wid * EPW
        pltpu.sync_copy(src_hbm.at[pl.ds(base, EPW)], src_v)
        pltpu.sync_copy(dst_hbm.at[pl.ds(base, EPW)], dst_v)
        iota = jnp.arange(16, dtype=jnp.int32)
        zeros = jnp.zeros((16,), jnp.int32)
        dump = jnp.full((16,), N, jnp.int32)

        def pre(i, _):
            for b in range(NB):
                sb_v[b][pl.ds(i * 16, 16)] = zeros
                db_v[b][pl.ds(i * 16, 16)] = dump
                eb_v[b][pl.ds(i * 16, 16)] = zeros
            return 0
        lax.fori_loop(0, PADB // 16, pre, 0)

        def body(j, curs):
            off = j * 16
            sv = src_v[pl.ds(off, 16)]
            dv = dst_v[pl.ds(off, 16)]
            valid = (iota + off) < EPW
            eid = base + off + iota
            q = ((dv >= NQ).astype(jnp.int32)
                 + (dv >= 2 * NQ).astype(jnp.int32)
                 + (dv >= 3 * NQ).astype(jnp.int32))
            new = []
            for b in range(NB):
                m = (q == b) & valid
                cs = jnp.cumsum(m.astype(jnp.int32))
                pos = curs[b] + cs - 1
                plsc.store_scatter(sb_v[b], [pos], sv, mask=m)
                plsc.store_scatter(db_v[b], [pos], dv, mask=m)
                plsc.store_scatter(eb_v[b], [pos], eid, mask=m)
                new.append(curs[b] + jnp.max(cs))
            return tuple(new)

        z32 = jnp.int32(0)
        curs = lax.fori_loop(0, (EPW + 15) // 16, body, (z32,) * NB)

        for b in range(NB):
            roff = (b * NW + wid) * PADB
            pltpu.sync_copy(sb_v[b], srcp_hbm.at[pl.ds(roff, PADB)])
            pltpu.sync_copy(db_v[b], dstl_hbm.at[pl.ds(roff, PADB)])
            pltpu.sync_copy(eb_v[b], eidp_hbm.at[pl.ds(roff, PADB)])
            cnt_v[...] = jnp.zeros((16,), jnp.int32) + curs[b]
            pltpu.sync_copy(cnt_v,
                            cnts_hbm.at[pl.ds((b * NW + wid) * 16, 16)])

    return k(src, dst)


def _sc_segsum(table, idxp, dstl, cnts, D):
    @functools.partial(
        pl.kernel,
        out_type=jax.ShapeDtypeStruct((NPAD, D), jnp.float32),
        mesh=_sc_mesh,
        compiler_params=_sc_params,
        scratch_types=[
            pltpu.VMEM((161, D), jnp.float32),
            pltpu.VMEM((PADB,), jnp.int32),
            pltpu.VMEM((PADB,), jnp.int32),
            pltpu.VMEM((PADB,), jnp.int32),
            pltpu.VMEM((PADB,), jnp.int32),
            pltpu.VMEM((CH, D), jnp.float32),
            pltpu.VMEM((CH, D), jnp.float32),
            pltpu.VMEM((16,), jnp.int32),
            pltpu.SemaphoreType.DMA,
            pltpu.SemaphoreType.DMA,
        ],
    )
    def k(table_hbm, idxp_hbm, dstl_hbm, cnts_hbm, out_hbm,
          acc, idx_v, dst_v, pend_i, pend_r, rows_a, rows_b, cntv,
          sema, semb):
        c = lax.axis_index("c")
        s = lax.axis_index("s")
        iota = jnp.arange(16, dtype=jnp.int32)
        zf = jnp.zeros((16,), jnp.float32)

        def pf(i, _):
            pend_i[pl.ds(i * 16, 16)] = jnp.zeros((16,), jnp.int32)
            pend_r[pl.ds(i * 16, 16)] = jnp.full((16,), 160, jnp.int32)
            return 0
        lax.fori_loop(0, PADB // 16, pf, 0)

        for half in range(2):
            b = c * 2 + half
            lo = s * 160

            def za(i, _):
                for jj in range(D // 16):
                    acc[i, pl.ds(jj * 16, 16)] = zf
                return 0
            lax.fori_loop(0, 161, za, 0)

            def region(r, _):
                roff = (b * NW + r) * PADB
                pltpu.sync_copy(
                    cnts_hbm.at[pl.ds((b * NW + r) * 16, 16)], cntv)
                cnt = jnp.max(cntv[...])

                npc = lax.shift_right_logical(cnt + 1023, 10)

                def cp(kk, _):
                    pltpu.sync_copy(
                        idxp_hbm.at[pl.ds(roff + kk * 1024, 1024)],
                        idx_v.at[pl.ds(kk * 1024, 1024)])
                    pltpu.sync_copy(
                        dstl_hbm.at[pl.ds(roff + kk * 1024, 1024)],
                        dst_v.at[pl.ds(kk * 1024, 1024)])
                    return 0
                lax.fori_loop(0, npc, cp, 0)

                nch = lax.shift_right_logical(cnt + 15, 4)

                def scan(j, pcur):
                    dv = dst_v[pl.ds(j * 16, 16)]
                    iv = idx_v[pl.ds(j * 16, 16)]
                    local = dv - b * NQ
                    m = (local >= lo) & (local < lo + 160)
                    cs = jnp.cumsum(m.astype(jnp.int32))
                    pos = pcur + cs - 1
                    plsc.store_scatter(pend_i, [pos], iv, mask=m)
                    plsc.store_scatter(pend_r, [pos], local - lo, mask=m)
                    return pcur + jnp.max(cs)
                pcur = lax.fori_loop(0, nch, scan, jnp.int32(0))

                nfl = jnp.maximum(
                    lax.shift_right_logical(pcur + CH - 1,
                                            CH.bit_length() - 1), 1)
                npr = lax.shift_right_logical(nfl + 1, 1)

                def accum(f, rows_v):
                    nacc = jnp.minimum(pcur - f * CH, CH)

                    def accb(i, _):
                        p = f * CH + i
                        grp = lax.shift_right_logical(p, 4) * 16
                        rv = pend_r[pl.ds(grp, 16)]
                        lane = p & 15
                        row = jnp.max(jnp.where(iota == lane, rv, 0))
                        for jj in range(D // 16):
                            plsc.addupdate(
                                acc.at[row, pl.ds(jj * 16, 16)],
                                rows_v[i, pl.ds(jj * 16, 16)])
                        return 0
                    lax.fori_loop(0, nacc, accb, 0)

                def pair(g, _):
                    f0 = 2 * g
                    f1 = f0 + 1
                    f1c = jnp.minimum(f1, nfl - 1)
                    cpa = pltpu.async_copy(
                        table_hbm.at[pend_i.at[pl.ds(f0 * CH, CH)]],
                        rows_a, sema)
                    cpb = pltpu.async_copy(
                        table_hbm.at[pend_i.at[pl.ds(f1c * CH, CH)]],
                        rows_b, semb)
                    cpa.wait()
                    accum(f0, rows_a)
                    cpb.wait()
                    accum(f1, rows_b)
                    return 0
                lax.fori_loop(0, npr, pair, 0)
                return 0
            lax.fori_loop(0, NW, region, 0)

            pltpu.sync_copy(acc.at[pl.ds(0, 160)],
                            out_hbm.at[pl.ds(b * ACC + lo, 160)])

    return k(table, idxp, dstl, cnts)


def _unpad_nodes(arr):
    return jnp.concatenate(
        [arr[b * ACC:b * ACC + NQ] for b in range(NB)], axis=0)


def _sc_gather(table, idx, DG):
    GC = 32
    NFULL = EPW // GC
    TAIL = EPW - NFULL * GC

    @functools.partial(
        pl.kernel,
        out_type=jax.ShapeDtypeStruct((E, DG), jnp.float32),
        mesh=_sc_mesh,
        compiler_params=_sc_params,
        scratch_types=[
            pltpu.VMEM((NFULL * GC + GC,), jnp.int32),
            pltpu.VMEM((GC, DG), jnp.float32),
            pltpu.VMEM((GC, DG), jnp.float32),
            pltpu.SemaphoreType.DMA,
            pltpu.SemaphoreType.DMA,
            pltpu.SemaphoreType.DMA,
            pltpu.SemaphoreType.DMA,
        ],
    )
    def k(table_hbm, idx_hbm, out_hbm, idx_v, rows_a, rows_b,
          sema, semb, semc, semd):
        c = lax.axis_index("c")
        s = lax.axis_index("s")
        base = (c * 16 + s) * EPW
        pltpu.sync_copy(idx_hbm.at[pl.ds(base, EPW)],
                        idx_v.at[pl.ds(0, EPW)])
        for t in range(GC // 16):
            idx_v[pl.ds(NFULL * GC + t * 16, 16)] = (
                jnp.zeros((16,), jnp.int32))
        pltpu.sync_copy(idx_hbm.at[pl.ds(base + NFULL * GC, TAIL)],
                        idx_v.at[pl.ds(NFULL * GC, TAIL)])

        def pair(g, _):
            f0 = 2 * g
            f1 = f0 + 1
            cpa = pltpu.async_copy(
                table_hbm.at[idx_v.at[pl.ds(f0 * GC, GC)]], rows_a, sema)
            cpb = pltpu.async_copy(
                table_hbm.at[idx_v.at[pl.ds(f1 * GC, GC)]], rows_b, semb)
            cpa.wait()
            wa = pltpu.async_copy(
                rows_a, out_hbm.at[pl.ds(base + f0 * GC, GC)], semc)
            cpb.wait()
            wb = pltpu.async_copy(
                rows_b, out_hbm.at[pl.ds(base + f1 * GC, GC)], semd)
            wa.wait()
            wb.wait()
            return 0
        lax.fori_loop(0, NFULL // 2, pair, 0)

        pltpu.async_copy(
            table_hbm.at[idx_v.at[pl.ds(NFULL * GC, GC)]], rows_a,
            sema).wait()
        pltpu.sync_copy(rows_a.at[pl.ds(0, TAIL)],
                        out_hbm.at[pl.ds(base + NFULL * GC, TAIL)])

    return k(table, idx)


def kernel(nv, ns, x, edge_index,
           Wms1, bms1, Wms2, bms2, Wmv, bmv, Wus1, bus1, Wus2, bus2):
    src = edge_index[0]
    dst = edge_index[1]

    phi = _mlp1(ns, Wms1, bms1, Wms2, bms2)

    WmvP = jnp.zeros((F, 3 * F), jnp.float32).at[0:L, :].set(Wmv)

    vnc0 = jnp.transpose(nv, (0, 2, 1)).reshape(N, 3 * F)

    phivn = jnp.concatenate([phi, vnc0], axis=1)
    phivjE = _sc_gather(phivn, src, 6 * F)
    xs = jnp.take(x, src, axis=0)
    xd = jnp.take(x, dst, axis=0)

    dva, ds_e = _edge_stage(phivjE, xs, xd, WmvP, bmv)

    dv = jax.ops.segment_sum(dva, dst, num_segments=N)
    ds = jax.ops.segment_sum(ds_e, dst, num_segments=N)
    vnc = vnc0 + dv
    s_new = ns + ds

    s2 = _mlp2(vnc, s_new, Wus1, bus1, Wus2, bus2)

    ones = jnp.ones((E,), jnp.float32)
    deg = jnp.maximum(jax.ops.segment_sum(ones, dst, num_segments=N), 1.0)
    srcp, dstl, eidp, cnts = _partition(src, dst)
    del eidp
    uv_sum = _unpad_nodes(_sc_segsum(vnc, srcp, dstl, cnts, 3 * F))
    s2_sum = _unpad_nodes(_sc_segsum(s2, srcp, dstl, cnts, 3 * F))

    vout_c, sout = _final(vnc, s_new, uv_sum, s2_sum, deg[:, None])
    vout = jnp.transpose(vout_c.reshape(N, 3, F), (0, 2, 1))
    return (vout, sout)
